# Initial kernel scaffold; baseline (speedup 1.0000x reference)
import jax
import jax.numpy as jnp
from jax.experimental import pallas as pl


def kernel(x, edge_index, W_msg, W_self, b):
    raise NotImplementedError("write your pallas kernel here")


# speedup vs baseline: 5.1085x; 5.1085x over previous
import functools

import jax
import jax.numpy as jnp
from jax import lax
from jax.experimental import pallas as pl
from jax.experimental.pallas import tpu as pltpu
from jax.experimental.pallas import tpu_sc as plsc

N = 10000
E = 320000
D = 128
DA = 144

NC = 2
NS = 16
NW = NC * NS

CH = 128
NCH = 79
E_PAD = NW * NCH * CH
ROWS_PER_TILE = N // NS

BLK = 2000


def _sc_body(xaug_hbm, src_hbm, dst_hbm, zeros_hbm, out_hbm,
             src_v, dst_v, rows_v, acc, sem):
  c = lax.axis_index("c")
  s = lax.axis_index("s")
  wid = c * NS + s
  pltpu.sync_copy(zeros_hbm, acc.at[pl.ds(s * ROWS_PER_TILE, ROWS_PER_TILE)])
  pltpu.sync_copy(src_hbm.at[pl.ds(wid * NCH, NCH)], src_v)
  pltpu.sync_copy(dst_hbm.at[pl.ds(wid * NCH, NCH)], dst_v)
  plsc.subcore_barrier()

  def body(j, carry):
    pltpu.async_copy(xaug_hbm.at[src_v.at[j]], rows_v, sem).wait()
    pltpu.sync_copy(rows_v, acc.at[dst_v.at[j]], add=True)
    return carry

  lax.fori_loop(0, NCH, body, 0)
  plsc.subcore_barrier()
  pltpu.sync_copy(acc.at[pl.ds(s * ROWS_PER_TILE, ROWS_PER_TILE)],
                  out_hbm.at[c, pl.ds(s * ROWS_PER_TILE, ROWS_PER_TILE)])


_sc_gather_scatter = functools.partial(
    pl.kernel,
    out_type=jax.ShapeDtypeStruct((NC, N, DA), jnp.float32),
    mesh=plsc.VectorSubcoreMesh(
        core_axis_name="c", subcore_axis_name="s", num_cores=NC,
        num_subcores=NS),
    scratch_types=[
        pltpu.VMEM((NCH, CH), jnp.int32),
        pltpu.VMEM((NCH, CH), jnp.int32),
        pltpu.VMEM((CH, DA), jnp.float32),
        pltpu.VMEM_SHARED((N, DA), jnp.float32),
        pltpu.SemaphoreType.DMA,
    ],
    compiler_params=pltpu.CompilerParams(use_tc_tiling_on_sc=False),
)(_sc_body)


def _tc_body(p_ref, x_ref, wm_ref, ws_ref, b_ref, o_ref):
  p = p_ref[0] + p_ref[1]
  deg = jnp.maximum(p[:, D:D + 1], 1.0)
  agg = p[:, :D] / deg
  h = jnp.dot(agg, wm_ref[...], preferred_element_type=jnp.float32)
  h = h + jnp.dot(x_ref[...], ws_ref[...], preferred_element_type=jnp.float32)
  h = h + b_ref[...]
  o_ref[...] = jnp.maximum(h, 0.0)


def kernel(x, edge_index, W_msg, W_self, b):
  xaug = jnp.zeros((N + 8, DA), jnp.float32)
  xaug = xaug.at[:N, :D].set(x).at[:N, D].set(1.0)
  src = edge_index[0].astype(jnp.int32)
  dst = edge_index[1].astype(jnp.int32)
  pad = E_PAD - E
  src_p = jnp.concatenate([src, jnp.full((pad,), N, jnp.int32)])
  dst_p = jnp.concatenate([dst, jnp.zeros((pad,), jnp.int32)])
  src_p = src_p.reshape(NW * NCH, CH)
  dst_p = dst_p.reshape(NW * NCH, CH)
  zeros_blk = jnp.zeros((ROWS_PER_TILE, DA), jnp.float32)

  partial = _sc_gather_scatter(xaug, src_p, dst_p, zeros_blk)

  out = pl.pallas_call(
      _tc_body,
      grid=(N // BLK,),
      in_specs=[
          pl.BlockSpec((NC, BLK, DA), lambda i: (0, i, 0)),
          pl.BlockSpec((BLK, D), lambda i: (i, 0)),
          pl.BlockSpec((D, D), lambda i: (0, 0)),
          pl.BlockSpec((D, D), lambda i: (0, 0)),
          pl.BlockSpec((1, D), lambda i: (0, 0)),
      ],
      out_specs=pl.BlockSpec((BLK, D), lambda i: (i, 0)),
      out_shape=jax.ShapeDtypeStruct((N, D), jnp.float32),
  )(partial, x, W_msg, W_self, b.reshape(1, D))
  return out
